# Initial kernel scaffold; baseline (speedup 1.0000x reference)
import jax
import jax.numpy as jnp
from jax.experimental import pallas as pl


def kernel(x, edge_index, W1, b1, W2, b2, W3, b3, Wfc, bfc):
    raise NotImplementedError("write your pallas kernel here")


# speedup vs baseline: 43.6899x; 43.6899x over previous
import functools

import jax
import jax.numpy as jnp
from jax import lax
from jax.experimental import pallas as pl
from jax.experimental.pallas import tpu as pltpu
from jax.experimental.pallas import tpu_sc as plsc

N = 10000
E = 320000
D = 128
NPAD = 10240
EPAD = 327680
NT = 16
CH = 128
K = EPAD // (NT * CH)
NPT = NPAD // NT

_mesh1 = plsc.VectorSubcoreMesh(core_axis_name="c", subcore_axis_name="s",
                                num_cores=1)
_mesh2 = plsc.VectorSubcoreMesh(core_axis_name="c", subcore_axis_name="s",
                                num_cores=2)


@functools.partial(
    pl.kernel,
    out_type=jax.ShapeDtypeStruct((NPAD,), jnp.float32),
    mesh=_mesh1,
    scratch_types=[
        pltpu.VMEM_SHARED((NPAD,), jnp.float32),
        pltpu.VMEM((K, CH), jnp.int32),
        pltpu.VMEM((CH,), jnp.float32),
    ],
)
def _sc_deg(dst_hbm, zeros_hbm, deg_hbm, deg_sh, dstb, onesb):
    s = lax.axis_index("s")
    rsl = pl.ds(s * NPT, NPT)
    pltpu.sync_copy(zeros_hbm.at[rsl], deg_sh.at[rsl])
    pltpu.sync_copy(dst_hbm.at[s], dstb)

    def fill(i, carry):
        onesb[pl.ds(i * 16, 16)] = jnp.full((16,), 1.0, jnp.float32)
        return carry

    lax.fori_loop(0, CH // 16, fill, 0)
    plsc.subcore_barrier()

    def body(j, carry):
        pltpu.sync_copy(onesb, deg_sh.at[dstb.at[j]], add=True)
        return carry

    lax.fori_loop(0, K, body, 0)
    plsc.subcore_barrier()
    pltpu.sync_copy(deg_sh.at[rsl], deg_hbm.at[rsl])


def _tc_b_body(x_ref, w1_ref, deg_ref, dis_ref, hh_ref):
    dis = lax.rsqrt(deg_ref[...] + 1.0)
    dis_ref[...] = dis
    h = jnp.dot(x_ref[...], w1_ref[...], preferred_element_type=jnp.float32)
    hh_ref[...] = h * dis


def _tc_b(x_p, W1, deg2):
    return pl.pallas_call(
        _tc_b_body,
        out_shape=(jax.ShapeDtypeStruct((NPAD, 1), jnp.float32),
                   jax.ShapeDtypeStruct((NPAD, 16), jnp.float32)),
    )(x_p, W1, deg2)


@functools.partial(
    pl.kernel,
    out_type=(jax.ShapeDtypeStruct((NPAD, 8), jnp.float32),
              jax.ShapeDtypeStruct((NPAD, 8), jnp.float32)),
    mesh=_mesh2,
    scratch_types=[
        pltpu.VMEM_SHARED((NPAD, 8), jnp.float32),
        pltpu.VMEM_SHARED((NPAD, 8), jnp.float32),
        pltpu.VMEM((K, CH), jnp.int32),
        pltpu.VMEM((K, CH), jnp.int32),
        pltpu.VMEM((CH, 8), jnp.float32),
    ],
)
def _sc_prop16(hh_lo, hh_hi, src_hbm, dst_hbm, zeros_hbm, acc_lo, acc_hi,
               hh_sh, acc_sh, srcb, dstb, rows):
    c = lax.axis_index("c")
    s = lax.axis_index("s")
    rsl = pl.ds(s * NPT, NPT)

    @pl.when(c == 0)
    def _():
        pltpu.sync_copy(hh_lo.at[rsl], hh_sh.at[rsl])

    @pl.when(c == 1)
    def _():
        pltpu.sync_copy(hh_hi.at[rsl], hh_sh.at[rsl])

    pltpu.sync_copy(zeros_hbm.at[rsl], acc_sh.at[rsl])
    pltpu.sync_copy(src_hbm.at[s], srcb)
    pltpu.sync_copy(dst_hbm.at[s], dstb)
    plsc.subcore_barrier()

    def body(j, carry):
        pltpu.sync_copy(hh_sh.at[srcb.at[j]], rows)
        pltpu.sync_copy(rows, acc_sh.at[dstb.at[j]], add=True)
        return carry

    lax.fori_loop(0, K, body, 0)
    plsc.subcore_barrier()

    @pl.when(c == 0)
    def _():
        pltpu.sync_copy(acc_sh.at[rsl], acc_lo.at[rsl])

    @pl.when(c == 1)
    def _():
        pltpu.sync_copy(acc_sh.at[rsl], acc_hi.at[rsl])


def _tc_d_body(acc_ref, hh_ref, dis_ref, b1_ref, w2_ref, b2_ref, w3_ref,
               b3_ref, wfc_ref, bfc_ref, u_ref, bc_ref):
    w3f = jnp.dot(w3_ref[...], wfc_ref[...],
                  preferred_element_type=jnp.float32)
    wc = jnp.dot(w2_ref[...], w3f, preferred_element_type=jnp.float32)
    beta = jnp.dot(b2_ref[...], w3f, preferred_element_type=jnp.float32)
    cc = jnp.dot(b3_ref[...], wfc_ref[...],
                 preferred_element_type=jnp.float32) + bfc_ref[...]
    dis = dis_ref[...]
    h1 = jnp.maximum((acc_ref[...] + hh_ref[...]) * dis + b1_ref[...], 0.0)
    u = jnp.dot(h1, wc, preferred_element_type=jnp.float32)
    u_ref[...] = u * dis
    bc_ref[...] = jnp.concatenate(
        [jnp.broadcast_to(beta, (4, 16)), jnp.broadcast_to(cc, (4, 16))],
        axis=0)


def _tc_d(acc, hh, dis2, b1, W2, b2, W3, b3, Wfc, bfc):
    return pl.pallas_call(
        _tc_d_body,
        out_shape=(jax.ShapeDtypeStruct((NPAD, 1), jnp.float32),
                   jax.ShapeDtypeStruct((8, 16), jnp.float32)),
    )(acc, hh, dis2, b1, W2, b2, W3, b3, Wfc, bfc)


@functools.partial(
    pl.kernel,
    out_type=jax.ShapeDtypeStruct((NPAD,), jnp.float32),
    mesh=_mesh1,
    scratch_types=[
        pltpu.VMEM_SHARED((NPAD,), jnp.float32),
        pltpu.VMEM_SHARED((NPAD,), jnp.float32),
        pltpu.VMEM_SHARED((NPAD,), jnp.float32),
        pltpu.VMEM((K, CH), jnp.int32),
        pltpu.VMEM((K, CH), jnp.int32),
        pltpu.VMEM((CH,), jnp.float32),
        pltpu.VMEM((NPT,), jnp.float32),
        pltpu.VMEM((NPT,), jnp.float32),
        pltpu.VMEM((NPT,), jnp.float32),
        pltpu.VMEM((16,), jnp.float32),
        pltpu.VMEM((16,), jnp.float32),
    ],
)
def _sc_prop1(u_hbm, dis_hbm, src_hbm, dst_hbm, zeros_hbm, beta_hbm, c_hbm,
              out_hbm, u_sh, a1_sh, a2_sh, srcb, dstb, vals, accw, uw, disw,
              betab, cb):
    s = lax.axis_index("s")
    rsl = pl.ds(s * NPT, NPT)
    pltpu.sync_copy(u_hbm.at[rsl], u_sh.at[rsl])
    pltpu.sync_copy(zeros_hbm.at[rsl], a1_sh.at[rsl])
    pltpu.sync_copy(zeros_hbm.at[rsl], a2_sh.at[rsl])
    pltpu.sync_copy(src_hbm.at[s], srcb)
    pltpu.sync_copy(dst_hbm.at[s], dstb)
    pltpu.sync_copy(dis_hbm.at[rsl], disw)
    pltpu.sync_copy(beta_hbm, betab)
    pltpu.sync_copy(c_hbm, cb)
    plsc.subcore_barrier()

    def body1(j, carry):
        pltpu.sync_copy(u_sh.at[srcb.at[j]], vals)
        pltpu.sync_copy(vals, a1_sh.at[dstb.at[j]], add=True)
        return carry

    lax.fori_loop(0, K, body1, 0)
    plsc.subcore_barrier()

    pltpu.sync_copy(a1_sh.at[rsl], accw)
    pltpu.sync_copy(u_sh.at[rsl], uw)

    def ew1(i, carry):
        ii = pl.ds(i * 16, 16)
        d = disw[ii]
        accw[ii] = ((accw[ii] + uw[ii]) * d + betab[...]) * d
        return carry

    lax.fori_loop(0, NPT // 16, ew1, 0)
    plsc.subcore_barrier()
    pltpu.sync_copy(accw, u_sh.at[rsl])
    plsc.subcore_barrier()

    def body2(j, carry):
        pltpu.sync_copy(u_sh.at[srcb.at[j]], vals)
        pltpu.sync_copy(vals, a2_sh.at[dstb.at[j]], add=True)
        return carry

    lax.fori_loop(0, K, body2, 0)
    plsc.subcore_barrier()

    pltpu.sync_copy(a2_sh.at[rsl], accw)
    pltpu.sync_copy(u_sh.at[rsl], uw)

    def ew2(i, carry):
        ii = pl.ds(i * 16, 16)
        uw[ii] = (accw[ii] + uw[ii]) * disw[ii] + cb[...]
        return carry

    lax.fori_loop(0, NPT // 16, ew2, 0)
    pltpu.sync_copy(uw, out_hbm.at[rsl])


def kernel(x, edge_index, W1, b1, W2, b2, W3, b3, Wfc, bfc):
    src = edge_index[0].astype(jnp.int32)
    dst = edge_index[1].astype(jnp.int32)
    x_p = jnp.pad(x, ((0, NPAD - N), (0, 0)))
    pad_idx = jnp.full((EPAD - E,), NPAD - 1, jnp.int32)
    src_p = jnp.concatenate([src, pad_idx]).reshape(NT, K, CH)
    dst_p = jnp.concatenate([dst, pad_idx]).reshape(NT, K, CH)
    zeros1 = jnp.zeros((NPAD,), jnp.float32)
    zeros8 = jnp.zeros((NPAD, 8), jnp.float32)

    deg = _sc_deg(dst_p, zeros1)
    dis2, hh = _tc_b(x_p, W1, deg.reshape(NPAD, 1))
    acc_lo, acc_hi = _sc_prop16(hh[:, :8], hh[:, 8:], src_p, dst_p, zeros8)
    acc = jnp.concatenate([acc_lo, acc_hi], axis=1)
    uhat, bc = _tc_d(acc, hh, dis2, b1.reshape(1, 16), W2, b2.reshape(1, 32),
                     W3, b3.reshape(1, 64), Wfc, bfc.reshape(1, 1))
    outv = _sc_prop1(uhat.reshape(NPAD), dis2.reshape(NPAD), src_p, dst_p,
                     zeros1, bc[0], bc[4])
    return outv[:N].reshape(N, 1)
